# baseline (speedup 1.0000x reference)
import functools

import jax
import jax.numpy as jnp
from jax import lax
from jax.experimental import pallas as pl
from jax.experimental.pallas import tpu as pltpu
from jax.experimental.pallas import tpu_sc as plsc

N = 10000
E = 320000
D = 128
DE = 16

NSC = 2
NSUB = 16
C = 128
MAX_CHUNKS = 160
TILE_CAP = MAX_CHUNKS * C
ECAP = NSUB * TILE_CAP
EPAD = ECAP
NPAD = 10240
HN = NPAD // NSC
HROWS = HN // NSUB
ACC_ROWS = HN + 8


def _sc_stage_body(feat_h, ep_h, src_h, dstl_h, eid_h, cnt_h, out_h,
                   src_v, dst_v, eid_v, cnt_v, xbuf, ebuf,
                   accum, isem, gsem, esem):
    c = lax.axis_index("c")
    s = lax.axis_index("s")
    pltpu.sync_copy(cnt_h, cnt_v.at[pl.ds(0, NSC * NSUB)])
    nk = cnt_v[pl.ds(c * NSUB + s, 16)][0]

    def zrow(r, carry):
        for j in range(D // 16):
            ebuf[0][r, pl.ds(j * 16, 16)] = jnp.zeros((16,), jnp.float32)
        return carry
    lax.fori_loop(0, C, zrow, 0)
    for t in range(HROWS // 64):
        pltpu.sync_copy(ebuf[0].at[pl.ds(0, 64)],
                        accum.at[pl.ds(s * HROWS + t * 64, 64)])
    @pl.when(s == 0)
    def _():
        pltpu.sync_copy(ebuf[0].at[pl.ds(0, 8)], accum.at[pl.ds(HN, 8)])
    plsc.subcore_barrier()

    edge0 = c * ECAP + s * TILE_CAP

    def issue_idx(k, p):
        base = edge0 + k * C
        pltpu.async_copy(src_h.at[pl.ds(base, C)], src_v[p], isem[p])
        pltpu.async_copy(dstl_h.at[pl.ds(base, C)], dst_v[p], isem[p])
        pltpu.async_copy(eid_h.at[pl.ds(base, C)], eid_v[p], isem[p])

    def wait_idx(k, p):
        base = edge0 + k * C
        pltpu.make_async_copy(src_h.at[pl.ds(base, C)], src_v[p], isem[p]).wait()
        pltpu.make_async_copy(dstl_h.at[pl.ds(base, C)], dst_v[p],
                              isem[p]).wait()
        pltpu.make_async_copy(eid_h.at[pl.ds(base, C)], eid_v[p],
                              isem[p]).wait()

    def issue_data(k, p):
        pltpu.async_copy(feat_h.at[src_v[p]], xbuf[p], gsem[p])
        pltpu.async_copy(ep_h.at[eid_v[p]], ebuf[p], esem[p])

    def wait_data(k, p):
        pltpu.make_async_copy(feat_h.at[src_v[p]], xbuf[p], gsem[p]).wait()
        pltpu.make_async_copy(ep_h.at[eid_v[p]], ebuf[p], esem[p]).wait()

    issue_idx(0, 0)
    issue_idx(1, 1)
    wait_idx(0, 0)
    issue_data(0, 0)

    def step(k, p, q):
        kp1 = k + 1
        @pl.when(kp1 < nk)
        def _():
            wait_idx(kp1, q)
            issue_data(kp1, q)
        wait_data(k, p)

        def crow(r, cc):
            for j in range(D // 16):
                sl = pl.ds(j * 16, 16)
                ebuf[p][r, sl] = jnp.maximum(xbuf[p][r, sl] + ebuf[p][r, sl], 0.0)
            return cc
        lax.fori_loop(0, C, crow, 0)

        pltpu.sync_copy(ebuf[p], accum.at[dst_v[p]], add=True)
        kp2 = k + 2
        @pl.when(kp2 < nk)
        def _():
            issue_idx(kp2, p)

    def pair(i, carry):
        step(2 * i, 0, 1)
        step(2 * i + 1, 1, 0)
        return carry
    lax.fori_loop(0, nk // 2, pair, 0)

    plsc.subcore_barrier()
    for t in range(HROWS // 64):
        r0 = s * HROWS + t * 64
        pltpu.sync_copy(accum.at[pl.ds(r0, 64)], ebuf[0].at[pl.ds(0, 64)])
        pltpu.sync_copy(ebuf[0].at[pl.ds(0, 64)],
                        out_h.at[pl.ds(c * HN + r0, 64)])


@jax.jit
def _sc_stage(feat, ep, src1d, dstl1d, eid1d, cnt):
    mesh = plsc.VectorSubcoreMesh(core_axis_name="c", subcore_axis_name="s")
    f = pl.kernel(
        _sc_stage_body,
        out_type=jax.ShapeDtypeStruct((NPAD, D), jnp.float32),
        mesh=mesh,
        scratch_types=[
            [pltpu.VMEM((C,), jnp.int32)] * 2,
            [pltpu.VMEM((C,), jnp.int32)] * 2,
            [pltpu.VMEM((C,), jnp.int32)] * 2,
            pltpu.VMEM((NSC * NSUB + 16,), jnp.int32),
            [pltpu.VMEM((C, D), jnp.float32)] * 2,
            [pltpu.VMEM((C, D), jnp.float32)] * 2,
            pltpu.VMEM_SHARED((ACC_ROWS, D), jnp.float32),
            [pltpu.SemaphoreType.DMA] * 2,
            [pltpu.SemaphoreType.DMA] * 2,
            [pltpu.SemaphoreType.DMA] * 2,
        ],
    )
    return f(feat, ep, src1d, dstl1d, eid1d, cnt)


_BE = 2048


def _eproj_body(ea_ref, w_ref, b_ref, *outs):
    ea = ea_ref[...]
    p = jnp.dot(ea, w_ref[...], preferred_element_type=jnp.float32) + b_ref[...]
    for i, o in enumerate(outs):
        o[...] = p[:, i * D:(i + 1) * D]


@jax.jit
def _eproj(edge_attr, wcat, bcat):
    grid = (EPAD // _BE,)
    return pl.pallas_call(
        _eproj_body,
        grid=grid,
        in_specs=[
            pl.BlockSpec((_BE, DE), lambda i: (i, 0)),
            pl.BlockSpec((DE, 4 * D), lambda i: (0, 0)),
            pl.BlockSpec((1, 4 * D), lambda i: (0, 0)),
        ],
        out_specs=[pl.BlockSpec((_BE, D), lambda i: (i, 0))] * 4,
        out_shape=[jax.ShapeDtypeStruct((EPAD, D), jnp.float32)] * 4,
    )(edge_attr, wcat, bcat)


_BN = 2000


def _mlp1_body(x_ref, agg_ref, wa_ref, ba_ref, wb_ref, bb_ref, eps_ref, h_ref):
    x = x_ref[...]
    eps = eps_ref[0, 0]
    hpre = (1.0 + eps) * x + agg_ref[...]
    t = jnp.maximum(jnp.dot(hpre, wa_ref[...], preferred_element_type=jnp.float32)
                    + ba_ref[...], 0.0)
    h = jnp.maximum(jnp.dot(t, wb_ref[...], preferred_element_type=jnp.float32)
                    + bb_ref[...], 0.0)
    h_ref[...] = h


@jax.jit
def _mlp1(x, agg, wa, ba, wb, bb, eps):
    grid = (N // _BN,)
    return pl.pallas_call(
        _mlp1_body,
        grid=grid,
        in_specs=[
            pl.BlockSpec((_BN, D), lambda i: (i, 0)),
            pl.BlockSpec((_BN, D), lambda i: (i, 0)),
            pl.BlockSpec((D, D), lambda i: (0, 0)),
            pl.BlockSpec((1, D), lambda i: (0, 0)),
            pl.BlockSpec((D, D), lambda i: (0, 0)),
            pl.BlockSpec((1, D), lambda i: (0, 0)),
            pl.BlockSpec((1, 1), lambda i: (0, 0)),
        ],
        out_specs=pl.BlockSpec((_BN, D), lambda i: (i, 0)),
        out_shape=jax.ShapeDtypeStruct((N, D), jnp.float32),
    )(x, agg, wa, ba, wb, bb, eps)


def _ln(z, g, b):
    mu = jnp.mean(z, axis=-1, keepdims=True)
    zc = z - mu
    var = jnp.mean(zc * zc, axis=-1, keepdims=True)
    return zc * jax.lax.rsqrt(var + 1e-5) * g + b


def _final_branch(h, agg, wa, ba, wb, bb, eps, g, b):
    h2 = (1.0 + eps) * h + agg
    t = jnp.maximum(jnp.dot(h2, wa, preferred_element_type=jnp.float32) + ba, 0.0)
    z = jax.nn.sigmoid(jnp.dot(t, wb, preferred_element_type=jnp.float32) + bb)
    return _ln(z, g, b)


def _final_body(hm_ref, aggm_ref, hs_ref, aggs_ref,
                wam_ref, bam_ref, wbm_ref, bbm_ref, epsm_ref,
                was_ref, bas_ref, wbs_ref, bbs_ref, epss_ref,
                g_ref, b_ref, noise_ref,
                samp_ref, mean_ref, std_ref):
    g = g_ref[...]
    b = b_ref[...]
    mean_v = _final_branch(hm_ref[...], aggm_ref[...], wam_ref[...], bam_ref[...],
                           wbm_ref[...], bbm_ref[...], epsm_ref[0, 0], g, b)
    std_v = _final_branch(hs_ref[...], aggs_ref[...], was_ref[...], bas_ref[...],
                          wbs_ref[...], bbs_ref[...], epss_ref[0, 0], g, b)
    mean_ref[...] = mean_v
    std_ref[...] = std_v
    samp_ref[...] = mean_v + noise_ref[...] * std_v


@jax.jit
def _final(hm, aggm, hs, aggs, wam, bam, wbm, bbm, epsm,
           was, bas, wbs, bbs, epss, g, b, noise):
    grid = (N // _BN,)
    row = lambda i: (i, 0)
    full = lambda i: (0, 0)
    return pl.pallas_call(
        _final_body,
        grid=grid,
        in_specs=[
            pl.BlockSpec((_BN, D), row),
            pl.BlockSpec((_BN, D), row),
            pl.BlockSpec((_BN, D), row),
            pl.BlockSpec((_BN, D), row),
            pl.BlockSpec((D, D), full), pl.BlockSpec((1, D), full),
            pl.BlockSpec((D, D), full), pl.BlockSpec((1, D), full),
            pl.BlockSpec((1, 1), full),
            pl.BlockSpec((D, D), full), pl.BlockSpec((1, D), full),
            pl.BlockSpec((D, D), full), pl.BlockSpec((1, D), full),
            pl.BlockSpec((1, 1), full),
            pl.BlockSpec((1, D), full), pl.BlockSpec((1, D), full),
            pl.BlockSpec((_BN, D), row),
        ],
        out_specs=[pl.BlockSpec((_BN, D), row)] * 3,
        out_shape=[jax.ShapeDtypeStruct((N, D), jnp.float32)] * 3,
    )(hm, aggm, hs, aggs, wam, bam, wbm, bbm, epsm,
      was, bas, wbs, bbs, epss, g, b, noise)


def kernel(x, edge_index, edge_attr, params):
    pm = params['mean']
    ps = params['std']
    src = edge_index[0].astype(jnp.int32)
    dst = edge_index[1].astype(jnp.int32)
    in0 = dst < HN
    pos0 = jnp.cumsum(in0) - 1
    pos1 = jnp.cumsum(~in0) - 1
    posx = jnp.where(in0, pos0, pos1)
    slot = ((posx % NSUB) * TILE_CAP + posx // NSUB
            + jnp.where(in0, 0, ECAP))
    src1d = jnp.zeros((2 * ECAP,), jnp.int32).at[slot].set(src)
    dstl1d = jnp.full((2 * ECAP,), HN, jnp.int32).at[slot].set(
        jnp.where(in0, dst, dst - HN))
    eid1d = jnp.full((2 * ECAP,), EPAD - 1, jnp.int32).at[slot].set(
        jnp.arange(E, dtype=jnp.int32))
    nc0 = jnp.sum(in0.astype(jnp.int32))
    s_arr = jnp.arange(NSUB, dtype=jnp.int32)
    cnts = []
    for nc in (nc0, E - nc0):
        nt = jnp.maximum(nc - s_arr + NSUB - 1, 0) // NSUB
        nk = (nt + C - 1) // C
        nk = jnp.maximum((nk + 1) // 2 * 2, 2)
        cnts.append(nk.astype(jnp.int32))
    cnt = jnp.concatenate(cnts)
    ea_p = jnp.concatenate([edge_attr,
                            jnp.zeros((EPAD - E, DE), jnp.float32)])

    wcat = jnp.concatenate([pm['We1'], pm['We2'], ps['We1'], ps['We2']], axis=1)
    bcat = jnp.concatenate([pm['be1'], pm['be2'], ps['be1'], ps['be2']])[None, :]
    ep1m, ep2m, ep1s, ep2s = _eproj(ea_p, wcat, bcat)

    r2 = lambda v: v[None, :]
    s2 = lambda v: v[None, None]

    agg1m = _sc_stage(x, ep1m, src1d, dstl1d, eid1d, cnt)
    hm = _mlp1(x, agg1m[:N], pm['W1a'], r2(pm['b1a']), pm['W1b'], r2(pm['b1b']),
               s2(pm['eps1']))
    agg2m = _sc_stage(hm, ep2m, src1d, dstl1d, eid1d, cnt)

    agg1s = _sc_stage(x, ep1s, src1d, dstl1d, eid1d, cnt)
    hs = _mlp1(x, agg1s[:N], ps['W1a'], r2(ps['b1a']), ps['W1b'], r2(ps['b1b']),
               s2(ps['eps1']))
    agg2s = _sc_stage(hs, ep2s, src1d, dstl1d, eid1d, cnt)

    noise = jax.random.normal(jax.random.key(42), (N, D), dtype=jnp.float32)
    samples, mean_v, std_v = _final(
        hm, agg2m[:N], hs, agg2s[:N],
        pm['W2a'], r2(pm['b2a']), pm['W2b'], r2(pm['b2b']), s2(pm['eps2']),
        ps['W2a'], r2(ps['b2a']), ps['W2b'], r2(ps['b2b']), s2(ps['eps2']),
        r2(params['ln_g']), r2(params['ln_b']), noise)
    return (samples, mean_v, std_v)


# speedup vs baseline: 1.4063x; 1.4063x over previous
import functools

import jax
import jax.numpy as jnp
from jax import lax
from jax.experimental import pallas as pl
from jax.experimental.pallas import tpu as pltpu
from jax.experimental.pallas import tpu_sc as plsc

N = 10000
E = 320000
D = 128
DE = 16

NSC = 2
NSUB = 16
C = 128
CHUNKS_PER_TILE = 160
EDGES_PER_TILE = CHUNKS_PER_TILE * C
EPAD = NSUB * EDGES_PER_TILE
NPAD = 10240
HN = NPAD // NSC
HROWS = HN // NSUB
ACC_ROWS = HN + 8


def _sc_stage_body(feat_h, ep_h, src_h, dstl_h, out_h,
                   src_v, dst_v, xbuf, ebuf,
                   accum, isem, gsem, esem):
    c = lax.axis_index("c")
    s = lax.axis_index("s")
    nk = CHUNKS_PER_TILE

    def zrow(r, carry):
        for j in range(D // 16):
            ebuf[0][r, pl.ds(j * 16, 16)] = jnp.zeros((16,), jnp.float32)
        return carry
    lax.fori_loop(0, C, zrow, 0)
    for t in range(HROWS // 64):
        pltpu.sync_copy(ebuf[0].at[pl.ds(0, 64)],
                        accum.at[pl.ds(s * HROWS + t * 64, 64)])
    @pl.when(s == 0)
    def _():
        pltpu.sync_copy(ebuf[0].at[pl.ds(0, 8)], accum.at[pl.ds(HN, 8)])
    plsc.subcore_barrier()

    edge0 = s * EDGES_PER_TILE
    dstl0 = c * EPAD + edge0

    def issue_idx(k, p):
        base = edge0 + k * C
        pltpu.async_copy(src_h.at[pl.ds(base, C)], src_v[p], isem[p])
        pltpu.async_copy(dstl_h.at[pl.ds(dstl0 + k * C, C)], dst_v[p], isem[p])

    def wait_idx(k, p):
        base = edge0 + k * C
        pltpu.make_async_copy(src_h.at[pl.ds(base, C)], src_v[p], isem[p]).wait()
        pltpu.make_async_copy(dstl_h.at[pl.ds(dstl0 + k * C, C)], dst_v[p],
                              isem[p]).wait()

    def issue_data(k, p):
        base = edge0 + k * C
        pltpu.async_copy(feat_h.at[src_v[p]], xbuf[p], gsem[p])
        pltpu.async_copy(ep_h.at[pl.ds(base, C)], ebuf[p], esem[p])

    def wait_data(k, p):
        base = edge0 + k * C
        pltpu.make_async_copy(feat_h.at[src_v[p]], xbuf[p], gsem[p]).wait()
        pltpu.make_async_copy(ep_h.at[pl.ds(base, C)], ebuf[p], esem[p]).wait()

    issue_idx(0, 0)
    issue_idx(1, 1)
    wait_idx(0, 0)
    issue_data(0, 0)

    def step(k, p, q):
        kp1 = k + 1
        @pl.when(kp1 < nk)
        def _():
            wait_idx(kp1, q)
            issue_data(kp1, q)
        wait_data(k, p)

        def crow(r, cc):
            for j in range(D // 16):
                sl = pl.ds(j * 16, 16)
                ebuf[p][r, sl] = jnp.maximum(xbuf[p][r, sl] + ebuf[p][r, sl], 0.0)
            return cc
        lax.fori_loop(0, C, crow, 0)

        pltpu.sync_copy(ebuf[p], accum.at[dst_v[p]], add=True)
        kp2 = k + 2
        @pl.when(kp2 < nk)
        def _():
            issue_idx(kp2, p)

    def pair(i, carry):
        step(2 * i, 0, 1)
        step(2 * i + 1, 1, 0)
        return carry
    lax.fori_loop(0, nk // 2, pair, 0)

    plsc.subcore_barrier()
    for t in range(HROWS // 64):
        r0 = s * HROWS + t * 64
        pltpu.sync_copy(accum.at[pl.ds(r0, 64)], ebuf[0].at[pl.ds(0, 64)])
        pltpu.sync_copy(ebuf[0].at[pl.ds(0, 64)],
                        out_h.at[pl.ds(c * HN + r0, 64)])


@jax.jit
def _sc_stage(feat, ep, src1d, dstl1d):
    mesh = plsc.VectorSubcoreMesh(core_axis_name="c", subcore_axis_name="s")
    f = pl.kernel(
        _sc_stage_body,
        out_type=jax.ShapeDtypeStruct((NPAD, D), jnp.float32),
        mesh=mesh,
        scratch_types=[
            [pltpu.VMEM((C,), jnp.int32)] * 2,
            [pltpu.VMEM((C,), jnp.int32)] * 2,
            [pltpu.VMEM((C, D), jnp.float32)] * 2,
            [pltpu.VMEM((C, D), jnp.float32)] * 2,
            pltpu.VMEM_SHARED((ACC_ROWS, D), jnp.float32),
            [pltpu.SemaphoreType.DMA] * 2,
            [pltpu.SemaphoreType.DMA] * 2,
            [pltpu.SemaphoreType.DMA] * 2,
        ],
    )
    return f(feat, ep, src1d, dstl1d)


_BE = 2048


def _eproj_body(ea_ref, w_ref, b_ref, *outs):
    ea = ea_ref[...]
    p = jnp.dot(ea, w_ref[...], preferred_element_type=jnp.float32) + b_ref[...]
    for i, o in enumerate(outs):
        o[...] = p[:, i * D:(i + 1) * D]


@jax.jit
def _eproj(edge_attr, wcat, bcat):
    grid = (EPAD // _BE,)
    return pl.pallas_call(
        _eproj_body,
        grid=grid,
        in_specs=[
            pl.BlockSpec((_BE, DE), lambda i: (i, 0)),
            pl.BlockSpec((DE, 4 * D), lambda i: (0, 0)),
            pl.BlockSpec((1, 4 * D), lambda i: (0, 0)),
        ],
        out_specs=[pl.BlockSpec((_BE, D), lambda i: (i, 0))] * 4,
        out_shape=[jax.ShapeDtypeStruct((EPAD, D), jnp.float32)] * 4,
    )(edge_attr, wcat, bcat)


_BN = 2000


def _mlp1_body(x_ref, agg_ref, wa_ref, ba_ref, wb_ref, bb_ref, eps_ref, h_ref):
    x = x_ref[...]
    eps = eps_ref[0, 0]
    hpre = (1.0 + eps) * x + agg_ref[...]
    t = jnp.maximum(jnp.dot(hpre, wa_ref[...], preferred_element_type=jnp.float32)
                    + ba_ref[...], 0.0)
    h = jnp.maximum(jnp.dot(t, wb_ref[...], preferred_element_type=jnp.float32)
                    + bb_ref[...], 0.0)
    h_ref[...] = h


@jax.jit
def _mlp1(x, agg, wa, ba, wb, bb, eps):
    grid = (N // _BN,)
    return pl.pallas_call(
        _mlp1_body,
        grid=grid,
        in_specs=[
            pl.BlockSpec((_BN, D), lambda i: (i, 0)),
            pl.BlockSpec((_BN, D), lambda i: (i, 0)),
            pl.BlockSpec((D, D), lambda i: (0, 0)),
            pl.BlockSpec((1, D), lambda i: (0, 0)),
            pl.BlockSpec((D, D), lambda i: (0, 0)),
            pl.BlockSpec((1, D), lambda i: (0, 0)),
            pl.BlockSpec((1, 1), lambda i: (0, 0)),
        ],
        out_specs=pl.BlockSpec((_BN, D), lambda i: (i, 0)),
        out_shape=jax.ShapeDtypeStruct((N, D), jnp.float32),
    )(x, agg, wa, ba, wb, bb, eps)


def _ln(z, g, b):
    mu = jnp.mean(z, axis=-1, keepdims=True)
    zc = z - mu
    var = jnp.mean(zc * zc, axis=-1, keepdims=True)
    return zc * jax.lax.rsqrt(var + 1e-5) * g + b


def _final_branch(h, agg, wa, ba, wb, bb, eps, g, b):
    h2 = (1.0 + eps) * h + agg
    t = jnp.maximum(jnp.dot(h2, wa, preferred_element_type=jnp.float32) + ba, 0.0)
    z = jax.nn.sigmoid(jnp.dot(t, wb, preferred_element_type=jnp.float32) + bb)
    return _ln(z, g, b)


def _final_body(hm_ref, aggm_ref, hs_ref, aggs_ref,
                wam_ref, bam_ref, wbm_ref, bbm_ref, epsm_ref,
                was_ref, bas_ref, wbs_ref, bbs_ref, epss_ref,
                g_ref, b_ref, noise_ref,
                samp_ref, mean_ref, std_ref):
    g = g_ref[...]
    b = b_ref[...]
    mean_v = _final_branch(hm_ref[...], aggm_ref[...], wam_ref[...], bam_ref[...],
                           wbm_ref[...], bbm_ref[...], epsm_ref[0, 0], g, b)
    std_v = _final_branch(hs_ref[...], aggs_ref[...], was_ref[...], bas_ref[...],
                          wbs_ref[...], bbs_ref[...], epss_ref[0, 0], g, b)
    mean_ref[...] = mean_v
    std_ref[...] = std_v
    samp_ref[...] = mean_v + noise_ref[...] * std_v


@jax.jit
def _final(hm, aggm, hs, aggs, wam, bam, wbm, bbm, epsm,
           was, bas, wbs, bbs, epss, g, b, noise):
    grid = (N // _BN,)
    row = lambda i: (i, 0)
    full = lambda i: (0, 0)
    return pl.pallas_call(
        _final_body,
        grid=grid,
        in_specs=[
            pl.BlockSpec((_BN, D), row),
            pl.BlockSpec((_BN, D), row),
            pl.BlockSpec((_BN, D), row),
            pl.BlockSpec((_BN, D), row),
            pl.BlockSpec((D, D), full), pl.BlockSpec((1, D), full),
            pl.BlockSpec((D, D), full), pl.BlockSpec((1, D), full),
            pl.BlockSpec((1, 1), full),
            pl.BlockSpec((D, D), full), pl.BlockSpec((1, D), full),
            pl.BlockSpec((D, D), full), pl.BlockSpec((1, D), full),
            pl.BlockSpec((1, 1), full),
            pl.BlockSpec((1, D), full), pl.BlockSpec((1, D), full),
            pl.BlockSpec((_BN, D), row),
        ],
        out_specs=[pl.BlockSpec((_BN, D), row)] * 3,
        out_shape=[jax.ShapeDtypeStruct((N, D), jnp.float32)] * 3,
    )(hm, aggm, hs, aggs, wam, bam, wbm, bbm, epsm,
      was, bas, wbs, bbs, epss, g, b, noise)


def kernel(x, edge_index, edge_attr, params):
    pm = params['mean']
    ps = params['std']
    src = edge_index[0].astype(jnp.int32)
    dst = edge_index[1].astype(jnp.int32)
    npad_e = EPAD - E
    src1d = jnp.concatenate([src, jnp.zeros((npad_e,), jnp.int32)])
    dst_p = jnp.concatenate([dst, jnp.full((npad_e,), NPAD, jnp.int32)])
    halves = []
    for cc in range(NSC):
        lo = cc * HN
        inr = (dst_p >= lo) & (dst_p < lo + HN)
        halves.append(jnp.where(inr, dst_p - lo, HN))
    dstl1d = jnp.concatenate(halves)
    ea_p = jnp.concatenate([edge_attr, jnp.zeros((npad_e, DE), jnp.float32)])

    wcat = jnp.concatenate([pm['We1'], pm['We2'], ps['We1'], ps['We2']], axis=1)
    bcat = jnp.concatenate([pm['be1'], pm['be2'], ps['be1'], ps['be2']])[None, :]
    ep1m, ep2m, ep1s, ep2s = _eproj(ea_p, wcat, bcat)

    r2 = lambda v: v[None, :]
    s2 = lambda v: v[None, None]

    agg1m = _sc_stage(x, ep1m, src1d, dstl1d)
    hm = _mlp1(x, agg1m[:N], pm['W1a'], r2(pm['b1a']), pm['W1b'], r2(pm['b1b']),
               s2(pm['eps1']))
    agg2m = _sc_stage(hm, ep2m, src1d, dstl1d)

    agg1s = _sc_stage(x, ep1s, src1d, dstl1d)
    hs = _mlp1(x, agg1s[:N], ps['W1a'], r2(ps['b1a']), ps['W1b'], r2(ps['b1b']),
               s2(ps['eps1']))
    agg2s = _sc_stage(hs, ep2s, src1d, dstl1d)

    noise = jax.random.normal(jax.random.key(42), (N, D), dtype=jnp.float32)
    samples, mean_v, std_v = _final(
        hm, agg2m[:N], hs, agg2s[:N],
        pm['W2a'], r2(pm['b2a']), pm['W2b'], r2(pm['b2b']), s2(pm['eps2']),
        ps['W2a'], r2(ps['b2a']), ps['W2b'], r2(ps['b2b']), s2(ps['eps2']),
        r2(params['ln_g']), r2(params['ln_b']), noise)
    return (samples, mean_v, std_v)
